# baseline (speedup 1.0000x reference)
import functools

import jax
import jax.numpy as jnp
from jax import lax
from jax.experimental import pallas as pl
from jax.experimental.pallas import tpu as pltpu
from jax.experimental.pallas import tpu_sc as plsc

N = 5000
NPAD = 5120
K = 100
KPAD = 112
D = 256
NI = 117
GRID = 50
IMG = 800.0
HUMAN_IDX = 1
NMS_THRESH = 0.5
L = 16
NSLC = KPAD // L


def _sread(ref, i):
    return ref[pl.ds(i, L)][0]


def _swrite(ref, i, val):
    idx = jnp.full((L,), i, jnp.int32)
    lane = lax.broadcasted_iota(jnp.int32, (L,), 0)
    plsc.store_scatter(ref, [idx], jnp.full((L,), val), mask=lane == 0)


def _sgather(ref, idx):
    return plsc.load_gather(ref, [jnp.full((L,), idx, jnp.int32)])[0]


def _sc_nms_gather(boxes_p, sc_u, lb_u, cell_u, order_p, feat):
    mesh = plsc.VectorSubcoreMesh(
        core_axis_name="c", subcore_axis_name="s", num_cores=2, num_subcores=16
    )

    @functools.partial(
        pl.kernel,
        out_type=[
            jax.ShapeDtypeStruct((KPAD, D), jnp.float32),
            jax.ShapeDtypeStruct((KPAD,), jnp.float32),
            jax.ShapeDtypeStruct((KPAD,), jnp.float32),
        ],
        mesh=mesh,
        scratch_types=[
            pltpu.VMEM((NPAD * 4,), jnp.float32),
            pltpu.VMEM((NPAD,), jnp.float32),
            pltpu.VMEM((NPAD,), jnp.int32),
            pltpu.VMEM((NPAD,), jnp.int32),
            pltpu.VMEM((NPAD,), jnp.int32),
            pltpu.VMEM((KPAD,), jnp.float32),
            pltpu.VMEM((KPAD,), jnp.float32),
            pltpu.VMEM((KPAD,), jnp.float32),
            pltpu.VMEM((KPAD,), jnp.float32),
            pltpu.VMEM((KPAD,), jnp.float32),
            pltpu.VMEM((KPAD,), jnp.float32),
            pltpu.VMEM((KPAD,), jnp.float32),
            pltpu.VMEM((KPAD,), jnp.int32),
            pltpu.VMEM((KPAD, D), jnp.float32),
            pltpu.SemaphoreType.DMA,
        ],
        compiler_params=pltpu.CompilerParams(needs_layout_passes=False),
    )
    def k(h_bx, h_sc, h_lb, h_cell, h_order, h_feat,
          o_f, o_ph, o_po,
          v_bx, v_sc, v_lb, v_cell_all, v_order,
          g_x1, g_y1, g_x2, g_y2, g_a, v_ph, v_po, v_cell, v_frows, sem):
        wid = lax.axis_index("s") * 2 + lax.axis_index("c")

        @pl.when(wid == 0)
        def _():
            cps = [
                pltpu.async_copy(h_bx, v_bx.at[pl.ds(0, N * 4)], sem),
                pltpu.async_copy(h_sc, v_sc.at[pl.ds(0, N)], sem),
                pltpu.async_copy(h_lb, v_lb.at[pl.ds(0, N)], sem),
                pltpu.async_copy(h_cell, v_cell_all.at[pl.ds(0, N)], sem),
                pltpu.async_copy(h_order, v_order.at[pl.ds(0, N)], sem),
            ]
            zf = jnp.zeros((L,), jnp.float32)
            zi = jnp.zeros((L,), jnp.int32)
            for g in range(NSLC):
                sl = pl.ds(g * L, L)
                g_x1[sl] = zf
                g_y1[sl] = zf
                g_x2[sl] = zf
                g_y2[sl] = zf
                g_a[sl] = zf
                v_ph[sl] = zf
                v_po[sl] = zf
                v_cell[sl] = zi
            for c in cps:
                c.wait()

            col = jnp.bitwise_and(lax.broadcasted_iota(jnp.int32, (L,), 0), 3)

            def cond(st):
                i, cnt = st
                return jnp.logical_and(i < N, cnt < K)

            def body(st):
                i, cnt = st
                idx = _sread(v_order, i)
                c4 = plsc.load_gather(
                    v_bx, [jnp.full((L,), idx * 4, jnp.int32) + col])
                lb = _sgather(v_lb, idx)
                off = lb.astype(jnp.float32) * (IMG + 2.0)
                x1 = c4[0] + off
                y1 = c4[1] + off
                x2 = c4[2] + off
                y2 = c4[3] + off
                a = (x2 - x1) * (y2 - y1)

                m = jnp.zeros((L,), jnp.float32)
                for g in range(NSLC):
                    sl = pl.ds(g * L, L)
                    ix = jnp.maximum(
                        jnp.minimum(x2, g_x2[sl]) - jnp.maximum(x1, g_x1[sl]), 0.0)
                    iy = jnp.maximum(
                        jnp.minimum(y2, g_y2[sl]) - jnp.maximum(y1, g_y1[sl]), 0.0)
                    inter = ix * iy
                    denom = a + g_a[sl] - inter + 1e-8
                    m = jnp.maximum(m, inter - NMS_THRESH * denom)
                supp = plsc.all_reduce_population_count(m > 0.0)[0] > 0

                @pl.when(jnp.logical_not(supp))
                def _acc():
                    _swrite(g_x1, cnt, x1)
                    _swrite(g_y1, cnt, y1)
                    _swrite(g_x2, cnt, x2)
                    _swrite(g_y2, cnt, y2)
                    _swrite(g_a, cnt, a)
                    s = _sgather(v_sc, idx)
                    _swrite(v_ph, cnt, jnp.where(lb == HUMAN_IDX, s, 0.0))
                    _swrite(v_po, cnt, s)
                    _swrite(v_cell, cnt, _sgather(v_cell_all, idx))

                return (i + 1, jnp.where(supp, cnt, cnt + 1))

            lax.while_loop(cond, body, (jnp.int32(0), jnp.int32(0)))

            pltpu.async_copy(h_feat.at[v_cell], v_frows, sem).wait()
            pltpu.sync_copy(v_frows, o_f)
            pltpu.sync_copy(v_ph, o_ph)
            pltpu.sync_copy(v_po, o_po)

    return k(boxes_p, sc_u, lb_u, cell_u, order_p, feat)


def _tc_pair(f, Wp, ph, po):
    NB = 4
    f4 = f[0:K].reshape(K // NB, NB, D)
    ph4 = ph[0:K].reshape(K // NB, NB, 1)
    po_row = po[0:K].reshape(1, K)

    def body(f4_ref, f_ref, wp_ref, ph_ref, po_ref, out_ref, b_ref):
        step = pl.program_id(0)

        @pl.when(step == 0)
        def _():
            b_ref[...] = jnp.dot(f_ref[...], wp_ref[D:2 * D, :],
                                 preferred_element_type=jnp.float32)

        A = jnp.dot(f4_ref[0], wp_ref[0:D, :],
                    preferred_element_type=jnp.float32)
        B = b_ref[0:K, :]
        logits = A[:, None, :] + B[None, :, :]
        prior = ph_ref[0] * po_ref[...]
        out_ref[...] = (1.0 / (1.0 + jnp.exp(-logits))) * prior[:, :, None]

    return pl.pallas_call(
        body,
        grid=(K // NB,),
        in_specs=[
            pl.BlockSpec((1, NB, D), lambda i: (i, 0, 0)),
            pl.BlockSpec((KPAD, D), lambda i: (0, 0)),
            pl.BlockSpec((2 * D, NI), lambda i: (0, 0)),
            pl.BlockSpec((1, NB, 1), lambda i: (i, 0, 0)),
            pl.BlockSpec((1, K), lambda i: (0, 0)),
        ],
        out_specs=pl.BlockSpec((NB, K, NI), lambda i: (i, 0, 0)),
        out_shape=jax.ShapeDtypeStruct((K, K, NI), jnp.float32),
        scratch_shapes=[pltpu.VMEM((KPAD, NI), jnp.float32)],
    )(f4, f, Wp, ph4, po_row)


def kernel(boxes, scores, labels, feat_map, Wp):
    order = jnp.argsort(-lax.stop_gradient(scores)).astype(jnp.int32)
    cx = lax.stop_gradient((boxes[:, 0] + boxes[:, 2]) * 0.5)
    cy = lax.stop_gradient((boxes[:, 1] + boxes[:, 3]) * 0.5)
    gx = jnp.clip((cx / IMG * GRID).astype(jnp.int32), 0, GRID - 1)
    gy = jnp.clip((cy / IMG * GRID).astype(jnp.int32), 0, GRID - 1)
    cell = gy * GRID + gx

    f, ph, po = _sc_nms_gather(
        boxes.reshape(-1), scores, labels, cell, order, feat_map)
    out3 = _tc_pair(f, Wp, ph, po)
    return out3.reshape(K * K, NI)


# speedup vs baseline: 1.1918x; 1.1918x over previous
import functools

import jax
import jax.numpy as jnp
from jax import lax
from jax.experimental import pallas as pl
from jax.experimental.pallas import tpu as pltpu
from jax.experimental.pallas import tpu_sc as plsc

N = 5000
NPAD = 5120
K = 100
KPAD = 112
D = 256
NI = 117
GRID = 50
IMG = 800.0
HUMAN_IDX = 1
NMS_THRESH = 0.5
L = 16
NSLC = KPAD // L


def _sread(ref, i):
    return ref[pl.ds(i, L)][0]


def _swrite(ref, i, val):
    idx = jnp.full((L,), i, jnp.int32)
    lane = lax.broadcasted_iota(jnp.int32, (L,), 0)
    plsc.store_scatter(ref, [idx], jnp.full((L,), val), mask=lane == 0)


def _sgather(ref, idx):
    return plsc.load_gather(ref, [jnp.full((L,), idx, jnp.int32)])[0]


def _sc_nms_gather(boxes_p, sc_u, lb_u, cell_u, order_p, feat):
    mesh = plsc.VectorSubcoreMesh(
        core_axis_name="c", subcore_axis_name="s", num_cores=2, num_subcores=16
    )

    @functools.partial(
        pl.kernel,
        out_type=[
            jax.ShapeDtypeStruct((KPAD, D), jnp.float32),
            jax.ShapeDtypeStruct((KPAD,), jnp.float32),
            jax.ShapeDtypeStruct((KPAD,), jnp.float32),
        ],
        mesh=mesh,
        scratch_types=[
            pltpu.VMEM((NPAD * 4,), jnp.float32),
            pltpu.VMEM((NPAD,), jnp.float32),
            pltpu.VMEM((NPAD,), jnp.int32),
            pltpu.VMEM((NPAD,), jnp.int32),
            pltpu.VMEM((NPAD,), jnp.int32),
            pltpu.VMEM((KPAD,), jnp.float32),
            pltpu.VMEM((KPAD,), jnp.float32),
            pltpu.VMEM((KPAD,), jnp.float32),
            pltpu.VMEM((KPAD,), jnp.float32),
            pltpu.VMEM((KPAD,), jnp.float32),
            pltpu.VMEM((KPAD,), jnp.float32),
            pltpu.VMEM((KPAD,), jnp.float32),
            pltpu.VMEM((KPAD,), jnp.int32),
            pltpu.VMEM((KPAD, D), jnp.float32),
            pltpu.SemaphoreType.DMA,
        ],
        compiler_params=pltpu.CompilerParams(needs_layout_passes=False),
    )
    def k(h_bx, h_sc, h_lb, h_cell, h_order, h_feat,
          o_f, o_ph, o_po,
          v_bx, v_sc, v_lb, v_cell_all, v_order,
          g_x1, g_y1, g_x2, g_y2, g_a, v_ph, v_po, v_cell, v_frows, sem):
        wid = lax.axis_index("s") * 2 + lax.axis_index("c")

        @pl.when(wid == 0)
        def _():
            cps = [
                pltpu.async_copy(h_bx, v_bx.at[pl.ds(0, N * 4)], sem),
                pltpu.async_copy(h_sc, v_sc.at[pl.ds(0, N)], sem),
                pltpu.async_copy(h_lb, v_lb.at[pl.ds(0, N)], sem),
                pltpu.async_copy(h_cell, v_cell_all.at[pl.ds(0, N)], sem),
                pltpu.async_copy(h_order, v_order.at[pl.ds(0, N)], sem),
            ]
            zf = jnp.zeros((L,), jnp.float32)
            zi = jnp.zeros((L,), jnp.int32)
            for g in range(NSLC):
                sl = pl.ds(g * L, L)
                g_x1[sl] = zf
                g_y1[sl] = zf
                g_x2[sl] = zf
                g_y2[sl] = zf
                g_a[sl] = zf
                v_ph[sl] = zf
                v_po[sl] = zf
                v_cell[sl] = zi
            for c in cps:
                c.wait()

            col = jnp.bitwise_and(lax.broadcasted_iota(jnp.int32, (L,), 0), 3)

            def cond(st):
                i, cnt = st
                return jnp.logical_and(i < N, cnt < K)

            def body(st):
                i, cnt = st
                idx = _sread(v_order, i)
                c4 = plsc.load_gather(
                    v_bx, [jnp.full((L,), idx * 4, jnp.int32) + col])
                lb = _sgather(v_lb, idx)
                off = lb.astype(jnp.float32) * (IMG + 2.0)
                x1 = c4[0] + off
                y1 = c4[1] + off
                x2 = c4[2] + off
                y2 = c4[3] + off
                a = (x2 - x1) * (y2 - y1)

                m = jnp.zeros((L,), jnp.float32)
                for g in range(NSLC):
                    sl = pl.ds(g * L, L)
                    ix = jnp.maximum(
                        jnp.minimum(x2, g_x2[sl]) - jnp.maximum(x1, g_x1[sl]), 0.0)
                    iy = jnp.maximum(
                        jnp.minimum(y2, g_y2[sl]) - jnp.maximum(y1, g_y1[sl]), 0.0)
                    inter = ix * iy
                    denom = a + g_a[sl] - inter + 1e-8
                    m = jnp.maximum(m, inter - NMS_THRESH * denom)
                supp = plsc.all_reduce_population_count(m > 0.0)[0] > 0

                @pl.when(jnp.logical_not(supp))
                def _acc():
                    _swrite(g_x1, cnt, x1)
                    _swrite(g_y1, cnt, y1)
                    _swrite(g_x2, cnt, x2)
                    _swrite(g_y2, cnt, y2)
                    _swrite(g_a, cnt, a)
                    s = _sgather(v_sc, idx)
                    _swrite(v_ph, cnt, jnp.where(lb == HUMAN_IDX, s, 0.0))
                    _swrite(v_po, cnt, s)
                    _swrite(v_cell, cnt, _sgather(v_cell_all, idx))

                return (i + 1, jnp.where(supp, cnt, cnt + 1))

            lax.while_loop(cond, body, (jnp.int32(0), jnp.int32(0)))

            pltpu.async_copy(h_feat.at[v_cell], v_frows, sem).wait()
            pltpu.sync_copy(v_frows, o_f)
            pltpu.sync_copy(v_ph, o_ph)
            pltpu.sync_copy(v_po, o_po)

    return k(boxes_p, sc_u, lb_u, cell_u, order_p, feat)


def _tc_pair(f, Wp, ph, po):
    NB = 20
    f4 = f[0:K].reshape(K // NB, NB, D)
    ph4 = ph[0:K].reshape(K // NB, NB, 1)
    po_row = po[0:K].reshape(1, K)

    def body(f4_ref, f_ref, wp_ref, ph_ref, po_ref, out_ref, b_ref):
        step = pl.program_id(0)

        @pl.when(step == 0)
        def _():
            b_ref[...] = jnp.dot(f_ref[...], wp_ref[D:2 * D, :],
                                 preferred_element_type=jnp.float32)

        A = jnp.dot(f4_ref[0], wp_ref[0:D, :],
                    preferred_element_type=jnp.float32)
        B = b_ref[0:K, :]
        logits = A[:, None, :] + B[None, :, :]
        prior = ph_ref[0] * po_ref[...]
        out_ref[...] = (1.0 / (1.0 + jnp.exp(-logits))) * prior[:, :, None]

    return pl.pallas_call(
        body,
        grid=(K // NB,),
        in_specs=[
            pl.BlockSpec((1, NB, D), lambda i: (i, 0, 0)),
            pl.BlockSpec((KPAD, D), lambda i: (0, 0)),
            pl.BlockSpec((2 * D, NI), lambda i: (0, 0)),
            pl.BlockSpec((1, NB, 1), lambda i: (i, 0, 0)),
            pl.BlockSpec((1, K), lambda i: (0, 0)),
        ],
        out_specs=pl.BlockSpec((NB, K, NI), lambda i: (i, 0, 0)),
        out_shape=jax.ShapeDtypeStruct((K, K, NI), jnp.float32),
        scratch_shapes=[pltpu.VMEM((KPAD, NI), jnp.float32)],
    )(f4, f, Wp, ph4, po_row)


def kernel(boxes, scores, labels, feat_map, Wp):
    order = jnp.argsort(-lax.stop_gradient(scores)).astype(jnp.int32)
    cx = lax.stop_gradient((boxes[:, 0] + boxes[:, 2]) * 0.5)
    cy = lax.stop_gradient((boxes[:, 1] + boxes[:, 3]) * 0.5)
    gx = jnp.clip((cx / IMG * GRID).astype(jnp.int32), 0, GRID - 1)
    gy = jnp.clip((cy / IMG * GRID).astype(jnp.int32), 0, GRID - 1)
    cell = gy * GRID + gx

    f, ph, po = _sc_nms_gather(
        boxes.reshape(-1), scores, labels, cell, order, feat_map)
    out3 = _tc_pair(f, Wp, ph, po)
    return out3.reshape(K * K, NI)


# speedup vs baseline: 1.2011x; 1.0078x over previous
import functools

import jax
import jax.numpy as jnp
from jax import lax
from jax.experimental import pallas as pl
from jax.experimental.pallas import tpu as pltpu
from jax.experimental.pallas import tpu_sc as plsc

N = 5000
NPAD = 5120
K = 100
KPAD = 112
D = 256
NI = 117
GRID = 50
IMG = 800.0
HUMAN_IDX = 1
NMS_THRESH = 0.5
L = 16
NSLC = KPAD // L
NB = 20


def _sread(ref, i):
    return ref[pl.ds(i, L)][0]


def _swrite(ref, i, val):
    idx = jnp.full((L,), i, jnp.int32)
    lane = lax.broadcasted_iota(jnp.int32, (L,), 0)
    plsc.store_scatter(ref, [idx], jnp.full((L,), val), mask=lane == 0)


def _sgather(ref, idx):
    return plsc.load_gather(ref, [jnp.full((L,), idx, jnp.int32)])[0]


def _sc_nms_gather(boxes_f, sc_u, lb_u, cell_u, order_v, feat):
    mesh = plsc.VectorSubcoreMesh(
        core_axis_name="c", subcore_axis_name="s", num_cores=2, num_subcores=16
    )

    @functools.partial(
        pl.kernel,
        out_type=[
            jax.ShapeDtypeStruct((KPAD, D), jnp.float32),
            jax.ShapeDtypeStruct((K,), jnp.float32),
            jax.ShapeDtypeStruct((K,), jnp.float32),
        ],
        mesh=mesh,
        scratch_types=[
            pltpu.VMEM((NPAD * 4,), jnp.float32),
            pltpu.VMEM((NPAD,), jnp.float32),
            pltpu.VMEM((NPAD,), jnp.int32),
            pltpu.VMEM((NPAD,), jnp.int32),
            pltpu.VMEM((NPAD,), jnp.int32),
            pltpu.VMEM((KPAD,), jnp.float32),
            pltpu.VMEM((KPAD,), jnp.float32),
            pltpu.VMEM((KPAD,), jnp.float32),
            pltpu.VMEM((KPAD,), jnp.float32),
            pltpu.VMEM((KPAD,), jnp.float32),
            pltpu.VMEM((KPAD,), jnp.float32),
            pltpu.VMEM((KPAD,), jnp.float32),
            pltpu.VMEM((KPAD,), jnp.int32),
            pltpu.VMEM((KPAD, D), jnp.float32),
            pltpu.SemaphoreType.DMA,
        ],
        compiler_params=pltpu.CompilerParams(needs_layout_passes=False),
    )
    def k(h_bx, h_sc, h_lb, h_cell, h_order, h_feat,
          o_f, o_ph, o_po,
          v_bx, v_sc, v_lb, v_cell_all, v_order,
          g_x1, g_y1, g_x2, g_y2, g_a, v_ph, v_po, v_cell, v_frows, sem):
        wid = lax.axis_index("s") * 2 + lax.axis_index("c")

        @pl.when(wid == 0)
        def _():
            cps = [
                pltpu.async_copy(h_bx, v_bx.at[pl.ds(0, N * 4)], sem),
                pltpu.async_copy(h_sc, v_sc.at[pl.ds(0, N)], sem),
                pltpu.async_copy(h_lb, v_lb.at[pl.ds(0, N)], sem),
                pltpu.async_copy(h_cell, v_cell_all.at[pl.ds(0, N)], sem),
                pltpu.async_copy(h_order, v_order.at[pl.ds(0, N)], sem),
            ]
            zf = jnp.zeros((L,), jnp.float32)
            zi = jnp.zeros((L,), jnp.int32)
            for g in range(NSLC):
                sl = pl.ds(g * L, L)
                g_x1[sl] = zf
                g_y1[sl] = zf
                g_x2[sl] = zf
                g_y2[sl] = zf
                g_a[sl] = zf
                v_ph[sl] = zf
                v_po[sl] = zf
                v_cell[sl] = zi
            for c in cps:
                c.wait()

            col = jnp.bitwise_and(lax.broadcasted_iota(jnp.int32, (L,), 0), 3)

            def fetch(i):
                idx = _sread(v_order, i)
                c4 = plsc.load_gather(
                    v_bx, [jnp.full((L,), idx * 4, jnp.int32) + col])
                lb = _sgather(v_lb, idx)
                return idx, c4, lb

            def cond(st):
                return jnp.logical_and(st[0] < N, st[1] < K)

            def body(st):
                i, cnt, idx, c4, lb = st
                idx_n, c4_n, lb_n = fetch(jnp.minimum(i + 1, N - 1))

                off = lb.astype(jnp.float32) * (IMG + 2.0)
                x1 = c4[0] + off
                y1 = c4[1] + off
                x2 = c4[2] + off
                y2 = c4[3] + off
                a = (x2 - x1) * (y2 - y1)

                m = jnp.zeros((L,), jnp.float32)
                for g in range(NSLC):
                    sl = pl.ds(g * L, L)
                    ix = jnp.maximum(
                        jnp.minimum(x2, g_x2[sl]) - jnp.maximum(x1, g_x1[sl]), 0.0)
                    iy = jnp.maximum(
                        jnp.minimum(y2, g_y2[sl]) - jnp.maximum(y1, g_y1[sl]), 0.0)
                    inter = ix * iy
                    denom = a + g_a[sl] - inter + 1e-8
                    m = jnp.maximum(m, inter - NMS_THRESH * denom)
                supp = plsc.all_reduce_population_count(m > 0.0)[0] > 0

                @pl.when(jnp.logical_not(supp))
                def _acc():
                    _swrite(g_x1, cnt, x1)
                    _swrite(g_y1, cnt, y1)
                    _swrite(g_x2, cnt, x2)
                    _swrite(g_y2, cnt, y2)
                    _swrite(g_a, cnt, a)
                    s = _sgather(v_sc, idx)
                    _swrite(v_ph, cnt, jnp.where(lb == HUMAN_IDX, s, 0.0))
                    _swrite(v_po, cnt, s)
                    _swrite(v_cell, cnt, _sgather(v_cell_all, idx))

                return (i + 1, jnp.where(supp, cnt, cnt + 1), idx_n, c4_n, lb_n)

            idx0, c40, lb0 = fetch(jnp.int32(0))
            lax.while_loop(cond, body,
                           (jnp.int32(0), jnp.int32(0), idx0, c40, lb0))

            pltpu.async_copy(h_feat.at[v_cell], v_frows, sem).wait()
            pltpu.sync_copy(v_frows, o_f)
            pltpu.sync_copy(v_ph.at[pl.ds(0, K)], o_ph)
            pltpu.sync_copy(v_po.at[pl.ds(0, K)], o_po)

    return k(boxes_f, sc_u, lb_u, cell_u, order_v, feat)


def _tc_pair(f, Wp, ph, po):
    f4 = f[0:K].reshape(K // NB, NB, D)
    ph4 = ph.reshape(K // NB, NB, 1)
    po_row = po.reshape(1, K)

    def body(f4_ref, f_ref, wp_ref, ph_ref, po_ref, out_ref, b_ref):
        step = pl.program_id(0)

        @pl.when(step == 0)
        def _():
            b_ref[...] = jnp.dot(f_ref[...], wp_ref[D:2 * D, :],
                                 preferred_element_type=jnp.float32)

        A = jnp.dot(f4_ref[0], wp_ref[0:D, :],
                    preferred_element_type=jnp.float32)
        B = b_ref[...][0:K, :]
        logits = A[:, None, :] + B[None, :, :]
        prior = ph_ref[0] * po_ref[...]
        out_ref[...] = (1.0 / (1.0 + jnp.exp(-logits))) * prior[:, :, None]

    return pl.pallas_call(
        body,
        grid=(K // NB,),
        in_specs=[
            pl.BlockSpec((1, NB, D), lambda i: (i, 0, 0)),
            pl.BlockSpec((KPAD, D), lambda i: (0, 0)),
            pl.BlockSpec((2 * D, NI), lambda i: (0, 0)),
            pl.BlockSpec((1, NB, 1), lambda i: (i, 0, 0)),
            pl.BlockSpec((1, K), lambda i: (0, 0)),
        ],
        out_specs=pl.BlockSpec((NB, K, NI), lambda i: (i, 0, 0)),
        out_shape=jax.ShapeDtypeStruct((K, K, NI), jnp.float32),
        scratch_shapes=[pltpu.VMEM((KPAD, NI), jnp.float32)],
    )(f4, f, Wp, ph4, po_row)


def kernel(boxes, scores, labels, feat_map, Wp):
    order = jnp.argsort(-lax.stop_gradient(scores)).astype(jnp.int32)
    cx = lax.stop_gradient((boxes[:, 0] + boxes[:, 2]) * 0.5)
    cy = lax.stop_gradient((boxes[:, 1] + boxes[:, 3]) * 0.5)
    gx = jnp.clip((cx / IMG * GRID).astype(jnp.int32), 0, GRID - 1)
    gy = jnp.clip((cy / IMG * GRID).astype(jnp.int32), 0, GRID - 1)
    cell = gy * GRID + gx

    f, ph, po = _sc_nms_gather(
        boxes.reshape(-1), scores, labels, cell, order, feat_map)
    out3 = _tc_pair(f, Wp, ph, po)
    return out3.reshape(K * K, NI)


# speedup vs baseline: 1.2287x; 1.0230x over previous
import functools

import jax
import jax.numpy as jnp
from jax import lax
from jax.experimental import pallas as pl
from jax.experimental.pallas import tpu as pltpu
from jax.experimental.pallas import tpu_sc as plsc

N = 5000
NPAD = 5120
K = 100
KPAD = 112
D = 256
NI = 117
GRID = 50
IMG = 800.0
HUMAN_IDX = 1
NMS_THRESH = 0.5
L = 16
NSLC = KPAD // L
NB = 20


def _sread(ref, i):
    return ref[pl.ds(i, L)][0]


def _swrite(ref, i, val):
    idx = jnp.full((L,), i, jnp.int32)
    lane = lax.broadcasted_iota(jnp.int32, (L,), 0)
    plsc.store_scatter(ref, [idx], jnp.full((L,), val), mask=lane == 0)


def _sgather(ref, idx):
    return plsc.load_gather(ref, [jnp.full((L,), idx, jnp.int32)])[0]


def _sc_nms_gather(boxes_f, sc_u, lb_u, cell_u, order_v, feat):
    mesh = plsc.VectorSubcoreMesh(
        core_axis_name="c", subcore_axis_name="s", num_cores=2, num_subcores=16
    )

    @functools.partial(
        pl.kernel,
        out_type=[
            jax.ShapeDtypeStruct((KPAD, D), jnp.float32),
            jax.ShapeDtypeStruct((K,), jnp.float32),
            jax.ShapeDtypeStruct((K,), jnp.float32),
        ],
        mesh=mesh,
        scratch_types=[
            pltpu.VMEM((NPAD * 4,), jnp.float32),
            pltpu.VMEM((NPAD,), jnp.float32),
            pltpu.VMEM((NPAD,), jnp.int32),
            pltpu.VMEM((NPAD,), jnp.int32),
            pltpu.VMEM((NPAD,), jnp.int32),
            pltpu.VMEM((KPAD,), jnp.float32),
            pltpu.VMEM((KPAD,), jnp.float32),
            pltpu.VMEM((KPAD,), jnp.float32),
            pltpu.VMEM((KPAD,), jnp.float32),
            pltpu.VMEM((KPAD,), jnp.float32),
            pltpu.VMEM((KPAD,), jnp.float32),
            pltpu.VMEM((KPAD,), jnp.float32),
            pltpu.VMEM((KPAD,), jnp.int32),
            pltpu.VMEM((KPAD, D), jnp.float32),
            pltpu.SemaphoreType.DMA,
        ],
        compiler_params=pltpu.CompilerParams(needs_layout_passes=False),
    )
    def k(h_bx, h_sc, h_lb, h_cell, h_order, h_feat,
          o_f, o_ph, o_po,
          v_bx, v_sc, v_lb, v_cell_all, v_order,
          g_x1, g_y1, g_x2, g_y2, g_a, v_ph, v_po, v_cell, v_frows, sem):
        wid = lax.axis_index("s") * 2 + lax.axis_index("c")

        @pl.when(wid == 0)
        def _():
            cps = [
                pltpu.async_copy(h_bx, v_bx.at[pl.ds(0, N * 4)], sem),
                pltpu.async_copy(h_sc, v_sc.at[pl.ds(0, N)], sem),
                pltpu.async_copy(h_lb, v_lb.at[pl.ds(0, N)], sem),
                pltpu.async_copy(h_cell, v_cell_all.at[pl.ds(0, N)], sem),
                pltpu.async_copy(h_order, v_order.at[pl.ds(0, N)], sem),
            ]
            zf = jnp.zeros((L,), jnp.float32)
            zi = jnp.zeros((L,), jnp.int32)
            for g in range(NSLC):
                sl = pl.ds(g * L, L)
                g_x1[sl] = zf
                g_y1[sl] = zf
                g_x2[sl] = zf
                g_y2[sl] = zf
                g_a[sl] = zf
                v_ph[sl] = zf
                v_po[sl] = zf
                v_cell[sl] = zi
            for c in cps:
                c.wait()

            col = jnp.bitwise_and(lax.broadcasted_iota(jnp.int32, (L,), 0), 3)

            def fetch(i):
                idx = _sread(v_order, i)
                c4 = plsc.load_gather(
                    v_bx, [jnp.full((L,), idx * 4, jnp.int32) + col])
                lb = _sgather(v_lb, idx)
                return idx, c4, lb

            def cond(st):
                return jnp.logical_and(st[0] < N, st[1] < K)

            def body(st):
                i, cnt, idx, c4, lb = st
                idx_n, c4_n, lb_n = fetch(jnp.minimum(i + 1, N - 1))

                off = lb.astype(jnp.float32) * (IMG + 2.0)
                x1 = c4[0] + off
                y1 = c4[1] + off
                x2 = c4[2] + off
                y2 = c4[3] + off
                a = (x2 - x1) * (y2 - y1)

                m = jnp.zeros((L,), jnp.float32)
                for g in range(NSLC):
                    sl = pl.ds(g * L, L)
                    ix = jnp.maximum(
                        jnp.minimum(x2, g_x2[sl]) - jnp.maximum(x1, g_x1[sl]), 0.0)
                    iy = jnp.maximum(
                        jnp.minimum(y2, g_y2[sl]) - jnp.maximum(y1, g_y1[sl]), 0.0)
                    inter = ix * iy
                    denom = a + g_a[sl] - inter + 1e-8
                    m = jnp.maximum(m, inter - NMS_THRESH * denom)
                supp = plsc.all_reduce_population_count(m > 0.0)[0] > 0

                @pl.when(jnp.logical_not(supp))
                def _acc():
                    _swrite(g_x1, cnt, x1)
                    _swrite(g_y1, cnt, y1)
                    _swrite(g_x2, cnt, x2)
                    _swrite(g_y2, cnt, y2)
                    _swrite(g_a, cnt, a)
                    s = _sgather(v_sc, idx)
                    _swrite(v_ph, cnt, jnp.where(lb == HUMAN_IDX, s, 0.0))
                    _swrite(v_po, cnt, s)
                    _swrite(v_cell, cnt, _sgather(v_cell_all, idx))

                return (i + 1, jnp.where(supp, cnt, cnt + 1), idx_n, c4_n, lb_n)

            idx0, c40, lb0 = fetch(jnp.int32(0))
            lax.while_loop(cond, body,
                           (jnp.int32(0), jnp.int32(0), idx0, c40, lb0))

            pltpu.async_copy(h_feat.at[v_cell], v_frows, sem).wait()
            pltpu.sync_copy(v_frows, o_f)
            pltpu.sync_copy(v_ph.at[pl.ds(0, K)], o_ph)
            pltpu.sync_copy(v_po.at[pl.ds(0, K)], o_po)

    return k(boxes_f, sc_u, lb_u, cell_u, order_v, feat)


def _tc_pair(f, Wp, ph, po):
    ph4 = ph.reshape(K // NB, NB, 1)
    po_row = po.reshape(1, K)

    def body(f_ref, wp_ref, ph_ref, po_ref, out_ref, a_ref, b_ref):
        step = pl.program_id(0)

        @pl.when(step == 0)
        def _():
            fv = f_ref[...]
            b_ref[...] = jnp.dot(fv, wp_ref[D:2 * D, :],
                                 preferred_element_type=jnp.float32)
            a_full = jnp.dot(fv, wp_ref[0:D, :],
                             preferred_element_type=jnp.float32)
            for t in range(K // NB):
                a_ref[t] = a_full[t * NB:(t + 1) * NB, :]

        A = a_ref[step]
        B = b_ref[...][0:K, :]
        logits = A[:, None, :] + B[None, :, :]
        prior = ph_ref[0] * po_ref[...]
        out_ref[...] = (1.0 / (1.0 + jnp.exp(-logits))) * prior[:, :, None]

    return pl.pallas_call(
        body,
        grid=(K // NB,),
        in_specs=[
            pl.BlockSpec((KPAD, D), lambda i: (0, 0)),
            pl.BlockSpec((2 * D, NI), lambda i: (0, 0)),
            pl.BlockSpec((1, NB, 1), lambda i: (i, 0, 0)),
            pl.BlockSpec((1, K), lambda i: (0, 0)),
        ],
        out_specs=pl.BlockSpec((NB, K, NI), lambda i: (i, 0, 0)),
        out_shape=jax.ShapeDtypeStruct((K, K, NI), jnp.float32),
        scratch_shapes=[pltpu.VMEM((K // NB, NB, NI), jnp.float32),
                        pltpu.VMEM((KPAD, NI), jnp.float32)],
    )(f, Wp, ph4, po_row)


def kernel(boxes, scores, labels, feat_map, Wp):
    order = jnp.argsort(-lax.stop_gradient(scores)).astype(jnp.int32)
    cx = lax.stop_gradient((boxes[:, 0] + boxes[:, 2]) * 0.5)
    cy = lax.stop_gradient((boxes[:, 1] + boxes[:, 3]) * 0.5)
    gx = jnp.clip((cx / IMG * GRID).astype(jnp.int32), 0, GRID - 1)
    gy = jnp.clip((cy / IMG * GRID).astype(jnp.int32), 0, GRID - 1)
    cell = gy * GRID + gx

    f, ph, po = _sc_nms_gather(
        boxes.reshape(-1), scores, labels, cell, order, feat_map)
    out3 = _tc_pair(f, Wp, ph, po)
    return out3.reshape(K * K, NI)


# speedup vs baseline: 1.2464x; 1.0144x over previous
import functools

import jax
import jax.numpy as jnp
from jax import lax
from jax.experimental import pallas as pl
from jax.experimental.pallas import tpu as pltpu
from jax.experimental.pallas import tpu_sc as plsc

N = 5000
NPAD = 5120
K = 100
KPAD = 112
D = 256
NI = 117
GRID = 50
IMG = 800.0
HUMAN_IDX = 1
NMS_THRESH = 0.5
L = 16
NSLC = KPAD // L
NB = 20


def _sread(ref, i):
    return ref[pl.ds(i, L)][0]


def _swrite(ref, i, val):
    idx = jnp.full((L,), i, jnp.int32)
    lane = lax.broadcasted_iota(jnp.int32, (L,), 0)
    plsc.store_scatter(ref, [idx], jnp.full((L,), val), mask=lane == 0)


def _sgather(ref, idx):
    return plsc.load_gather(ref, [jnp.full((L,), idx, jnp.int32)])[0]


def _sc_nms_gather(boxes_f, sc_u, lb_u, cell_u, order_v, feat):
    mesh = plsc.VectorSubcoreMesh(
        core_axis_name="c", subcore_axis_name="s", num_cores=2, num_subcores=16
    )

    @functools.partial(
        pl.kernel,
        out_type=[
            jax.ShapeDtypeStruct((KPAD, D), jnp.float32),
            jax.ShapeDtypeStruct((K,), jnp.float32),
            jax.ShapeDtypeStruct((K,), jnp.float32),
        ],
        mesh=mesh,
        scratch_types=[
            pltpu.VMEM((NPAD * 4,), jnp.float32),
            pltpu.VMEM((NPAD,), jnp.float32),
            pltpu.VMEM((NPAD,), jnp.int32),
            pltpu.VMEM((NPAD,), jnp.int32),
            pltpu.VMEM((NPAD,), jnp.int32),
            pltpu.VMEM((KPAD,), jnp.float32),
            pltpu.VMEM((KPAD,), jnp.float32),
            pltpu.VMEM((KPAD,), jnp.float32),
            pltpu.VMEM((KPAD,), jnp.float32),
            pltpu.VMEM((KPAD,), jnp.float32),
            pltpu.VMEM((KPAD,), jnp.float32),
            pltpu.VMEM((KPAD,), jnp.float32),
            pltpu.VMEM((KPAD,), jnp.int32),
            pltpu.VMEM((KPAD,), jnp.int32),
            pltpu.VMEM((KPAD, D), jnp.float32),
            pltpu.SemaphoreType.DMA,
        ],
        compiler_params=pltpu.CompilerParams(needs_layout_passes=False),
    )
    def k(h_bx, h_sc, h_lb, h_cell, h_order, h_feat,
          o_f, o_ph, o_po,
          v_bx, v_sc, v_lb, v_cell_all, v_order,
          g_x1, g_y1, g_x2, g_y2, g_a, v_ph, v_po, v_cell, v_kept, v_frows,
          sem):
        wid = lax.axis_index("s") * 2 + lax.axis_index("c")

        @pl.when(wid == 0)
        def _():
            cps = [
                pltpu.async_copy(h_bx, v_bx.at[pl.ds(0, N * 4)], sem),
                pltpu.async_copy(h_sc, v_sc.at[pl.ds(0, N)], sem),
                pltpu.async_copy(h_lb, v_lb.at[pl.ds(0, N)], sem),
                pltpu.async_copy(h_cell, v_cell_all.at[pl.ds(0, N)], sem),
                pltpu.async_copy(h_order, v_order.at[pl.ds(0, N)], sem),
            ]
            zf = jnp.zeros((L,), jnp.float32)
            zi = jnp.zeros((L,), jnp.int32)
            for g in range(NSLC):
                sl = pl.ds(g * L, L)
                g_x1[sl] = zf
                g_y1[sl] = zf
                g_x2[sl] = zf
                g_y2[sl] = zf
                g_a[sl] = zf
                v_ph[sl] = zf
                v_po[sl] = zf
                v_cell[sl] = zi
                v_kept[sl] = zi
            for c in cps:
                c.wait()

            col = jnp.bitwise_and(lax.broadcasted_iota(jnp.int32, (L,), 0), 3)

            def fetch(i):
                idx = _sread(v_order, i)
                c4 = plsc.load_gather(
                    v_bx, [jnp.full((L,), idx * 4, jnp.int32) + col])
                lb = _sgather(v_lb, idx)
                return idx, c4, lb

            def cond(st):
                return jnp.logical_and(st[0] < N, st[1] < K)

            def body(st):
                i, cnt, idx, c4, lb = st
                idx_n, c4_n, lb_n = fetch(jnp.minimum(i + 1, N - 1))

                off = lb.astype(jnp.float32) * (IMG + 2.0)
                x1 = c4[0] + off
                y1 = c4[1] + off
                x2 = c4[2] + off
                y2 = c4[3] + off
                a = (x2 - x1) * (y2 - y1)

                m = jnp.zeros((L,), jnp.float32)
                for g in range(NSLC):
                    sl = pl.ds(g * L, L)
                    ix = jnp.maximum(
                        jnp.minimum(x2, g_x2[sl]) - jnp.maximum(x1, g_x1[sl]), 0.0)
                    iy = jnp.maximum(
                        jnp.minimum(y2, g_y2[sl]) - jnp.maximum(y1, g_y1[sl]), 0.0)
                    inter = ix * iy
                    denom = a + g_a[sl] - inter + 1e-8
                    m = jnp.maximum(m, inter - NMS_THRESH * denom)
                supp = plsc.all_reduce_population_count(m > 0.0)[0] > 0

                @pl.when(jnp.logical_not(supp))
                def _acc():
                    _swrite(g_x1, cnt, x1)
                    _swrite(g_y1, cnt, y1)
                    _swrite(g_x2, cnt, x2)
                    _swrite(g_y2, cnt, y2)
                    _swrite(g_a, cnt, a)
                    _swrite(v_kept, cnt, idx)

                return (i + 1, jnp.where(supp, cnt, cnt + 1), idx_n, c4_n, lb_n)

            idx0, c40, lb0 = fetch(jnp.int32(0))
            fin = lax.while_loop(cond, body,
                                 (jnp.int32(0), jnp.int32(0), idx0, c40, lb0))
            cnt_fin = fin[1]

            for g in range(NSLC):
                sl = pl.ds(g * L, L)
                ki = v_kept[sl]
                kmask = (lax.broadcasted_iota(jnp.int32, (L,), 0)
                         + g * L) < cnt_fin
                s = plsc.load_gather(v_sc, [ki])
                lbv = plsc.load_gather(v_lb, [ki])
                sz = jnp.where(kmask, s, 0.0)
                v_po[sl] = sz
                v_ph[sl] = jnp.where(lbv == HUMAN_IDX, sz, 0.0)
                v_cell[sl] = plsc.load_gather(v_cell_all, [ki])

            pltpu.async_copy(h_feat.at[v_cell], v_frows, sem).wait()
            pltpu.sync_copy(v_frows, o_f)
            pltpu.sync_copy(v_ph.at[pl.ds(0, K)], o_ph)
            pltpu.sync_copy(v_po.at[pl.ds(0, K)], o_po)

    return k(boxes_f, sc_u, lb_u, cell_u, order_v, feat)


def _tc_pair(f, Wp, ph, po):
    ph4 = ph.reshape(K // NB, NB, 1)
    po_row = po.reshape(1, K)

    def body(f_ref, wp_ref, ph_ref, po_ref, out_ref, a_ref, b_ref):
        step = pl.program_id(0)

        @pl.when(step == 0)
        def _():
            fv = f_ref[...]
            b_ref[...] = jnp.dot(fv, wp_ref[D:2 * D, :],
                                 preferred_element_type=jnp.float32)
            a_full = jnp.dot(fv, wp_ref[0:D, :],
                             preferred_element_type=jnp.float32)
            for t in range(K // NB):
                a_ref[t] = a_full[t * NB:(t + 1) * NB, :]

        A = a_ref[step]
        B = b_ref[...][0:K, :]
        logits = A[:, None, :] + B[None, :, :]
        prior = ph_ref[0] * po_ref[...]
        out_ref[...] = (1.0 / (1.0 + jnp.exp(-logits))) * prior[:, :, None]

    return pl.pallas_call(
        body,
        grid=(K // NB,),
        in_specs=[
            pl.BlockSpec((KPAD, D), lambda i: (0, 0)),
            pl.BlockSpec((2 * D, NI), lambda i: (0, 0)),
            pl.BlockSpec((1, NB, 1), lambda i: (i, 0, 0)),
            pl.BlockSpec((1, K), lambda i: (0, 0)),
        ],
        out_specs=pl.BlockSpec((NB, K, NI), lambda i: (i, 0, 0)),
        out_shape=jax.ShapeDtypeStruct((K, K, NI), jnp.float32),
        scratch_shapes=[pltpu.VMEM((K // NB, NB, NI), jnp.float32),
                        pltpu.VMEM((KPAD, NI), jnp.float32)],
    )(f, Wp, ph4, po_row)


def kernel(boxes, scores, labels, feat_map, Wp):
    order = jnp.argsort(-lax.stop_gradient(scores)).astype(jnp.int32)
    cx = lax.stop_gradient((boxes[:, 0] + boxes[:, 2]) * 0.5)
    cy = lax.stop_gradient((boxes[:, 1] + boxes[:, 3]) * 0.5)
    gx = jnp.clip((cx / IMG * GRID).astype(jnp.int32), 0, GRID - 1)
    gy = jnp.clip((cy / IMG * GRID).astype(jnp.int32), 0, GRID - 1)
    cell = gy * GRID + gx

    f, ph, po = _sc_nms_gather(
        boxes.reshape(-1), scores, labels, cell, order, feat_map)
    out3 = _tc_pair(f, Wp, ph, po)
    return out3.reshape(K * K, NI)
